# baseline (speedup 1.0000x reference)
import functools

import jax
import jax.numpy as jnp
from jax import lax
from jax.experimental import pallas as pl
from jax.experimental.pallas import tpu as pltpu
from jax.experimental.pallas import tpu_sc as plsc

CH = 128
NBUF = 8
K = 6


@functools.lru_cache(maxsize=None)
def _make_gather(n_rows, V, D):
    info = plsc.get_sparse_core_info()
    NC, NS = info.num_cores, info.num_subcores
    NW = NC * NS
    n_per_w = n_rows // NW
    n_chunks = n_per_w
    assert n_rows % NW == 0 and n_chunks % NBUF == 0

    mesh = plsc.VectorSubcoreMesh(core_axis_name="c", subcore_axis_name="s")

    scratch = [pltpu.VMEM((n_chunks, CH), jnp.int32)]
    scratch += [pltpu.VMEM((CH, D), jnp.float32) for _ in range(NBUF)]
    scratch += [pltpu.SemaphoreType.DMA for _ in range(2 * NBUF)]

    @functools.partial(
        pl.kernel,
        mesh=mesh,
        out_type=jax.ShapeDtypeStruct((n_rows * CH, D), jnp.float32),
        scratch_types=scratch,
        compiler_params=pltpu.CompilerParams(use_tc_tiling_on_sc=False),
    )
    def k(tok_hbm, table_hbm, out_hbm, idx_v, *rest):
        bufs = rest[:NBUF]
        sg = rest[NBUF:2 * NBUF]
        so = rest[2 * NBUF:3 * NBUF]
        wid = lax.axis_index("s") * NC + lax.axis_index("c")
        row0 = wid * n_chunks

        pltpu.sync_copy(tok_hbm.at[pl.ds(row0, n_chunks)], idx_v)

        def gather_desc(c, b):
            return pltpu.make_async_copy(
                table_hbm.at[idx_v.at[c]], bufs[b], sg[b])

        def out_desc(c, b):
            return pltpu.make_async_copy(
                bufs[b], out_hbm.at[pl.ds((row0 + c) * CH, CH)], so[b])

        for b in range(K):
            gather_desc(b, b).start()

        def body(i, carry):
            for b in range(NBUF):
                c = i * NBUF + b
                gather_desc(c, b).wait()
                out_desc(c, b).start()
                cn = c + K
                bn = (b + K) % NBUF

                @pl.when(cn < n_chunks)
                def _issue():
                    @pl.when(cn >= NBUF)
                    def _drain():
                        out_desc(cn - NBUF, bn).wait()
                    gather_desc(cn, bn).start()
            return carry

        lax.fori_loop(0, n_chunks // NBUF, body, 0)

        for b in range(NBUF):
            out_desc(n_chunks - NBUF + b, b).wait()

    return k


def kernel(tokens, embeddings):
    Bt, T = tokens.shape
    V, D = embeddings.shape
    n_flat = Bt * T
    tok = tokens.reshape(n_flat // CH, CH).astype(jnp.int32)
    out = _make_gather(n_flat // CH, V, D)(tok, embeddings)
    return out.reshape(Bt, T, D)
